# Initial kernel scaffold; baseline (speedup 1.0000x reference)
import jax
import jax.numpy as jnp
from jax.experimental import pallas as pl


def kernel(hidden_states, router_weight, router_bias, expert_bias):
    raise NotImplementedError("write your pallas kernel here")


# speedup vs baseline: 10.0856x; 10.0856x over previous
import jax
import jax.numpy as jnp
from jax.experimental import pallas as pl

_E = 64
_ROWS = 512


def _fused_body(x_ref, wt_ref, rb_ref, eb_ref, out_ref):
    x = x_ref[...]
    logits = jnp.dot(x, wt_ref[...], preferred_element_type=jnp.float32)
    logits = logits + rb_ref[...]
    m = jnp.max(logits, axis=-1, keepdims=True)
    ex = jnp.exp(logits - m)
    scores = ex / jnp.sum(ex, axis=-1, keepdims=True)

    idx = jax.lax.broadcasted_iota(jnp.int32, scores.shape, 1)
    m1 = jnp.max(scores, axis=-1, keepdims=True)
    i1 = jnp.min(jnp.where(scores == m1, idx, _E), axis=-1, keepdims=True)
    s2 = jnp.where(idx == i1, -jnp.inf, scores)
    m2 = jnp.max(s2, axis=-1, keepdims=True)
    i2 = jnp.min(jnp.where(s2 == m2, idx, _E), axis=-1, keepdims=True)
    keep = (idx == i1) | (idx == i2)
    s_masked = jnp.where(keep, scores, 0.0)

    comb = jnp.dot(s_masked, eb_ref[...], preferred_element_type=jnp.float32)
    out_ref[...] = (m1 + m2) * x + comb


def kernel(hidden_states, router_weight, router_bias, expert_bias):
    B, S, H = hidden_states.shape
    BS = B * S
    flat = hidden_states.reshape(BS, H)
    wt = router_weight.T
    rb = router_bias.reshape(1, _E)

    out = pl.pallas_call(
        _fused_body,
        grid=(BS // _ROWS,),
        in_specs=[
            pl.BlockSpec((_ROWS, H), lambda i: (i, 0)),
            pl.BlockSpec((H, _E), lambda i: (0, 0)),
            pl.BlockSpec((1, _E), lambda i: (0, 0)),
            pl.BlockSpec((_E, H), lambda i: (0, 0)),
        ],
        out_specs=pl.BlockSpec((_ROWS, H), lambda i: (i, 0)),
        out_shape=jax.ShapeDtypeStruct((BS, H), jnp.float32),
    )(flat, wt, rb, expert_bias)
    return out.reshape(B, S, H)


# speedup vs baseline: 10.2785x; 1.0191x over previous
import jax
import jax.numpy as jnp
from jax.experimental import pallas as pl

_E = 64
_ROWS = 512


def _fused_body(x_ref, wt_ref, rb_ref, eb_ref, out_ref):
    x = x_ref[...]
    logits = jnp.dot(x, wt_ref[...], preferred_element_type=jnp.float32)
    logits = logits + rb_ref[...]
    m = jnp.max(logits, axis=-1, keepdims=True)
    ex = jnp.exp(logits - m)
    scores = ex / jnp.sum(ex, axis=-1, keepdims=True)

    m1 = jnp.max(scores, axis=-1, keepdims=True)
    s2 = jnp.where(scores == m1, -jnp.inf, scores)
    m2 = jnp.max(s2, axis=-1, keepdims=True)
    s_masked = jnp.where(scores >= m2, scores, 0.0)

    comb = jnp.dot(s_masked.astype(jnp.bfloat16), eb_ref[...],
                   preferred_element_type=jnp.float32)
    out_ref[...] = (m1 + m2) * x + comb


def kernel(hidden_states, router_weight, router_bias, expert_bias):
    B, S, H = hidden_states.shape
    BS = B * S
    flat = hidden_states.reshape(BS, H)
    wt = router_weight.T
    rb = router_bias.reshape(1, _E)
    eb16 = expert_bias.astype(jnp.bfloat16)

    out = pl.pallas_call(
        _fused_body,
        grid=(BS // _ROWS,),
        in_specs=[
            pl.BlockSpec((_ROWS, H), lambda i: (i, 0)),
            pl.BlockSpec((H, _E), lambda i: (0, 0)),
            pl.BlockSpec((1, _E), lambda i: (0, 0)),
            pl.BlockSpec((_E, H), lambda i: (0, 0)),
        ],
        out_specs=pl.BlockSpec((_ROWS, H), lambda i: (i, 0)),
        out_shape=jax.ShapeDtypeStruct((BS, H), jnp.float32),
    )(flat, wt, rb, eb16)
    return out.reshape(B, S, H)


# speedup vs baseline: 11.2446x; 1.0940x over previous
import jax
import jax.numpy as jnp
from jax.experimental import pallas as pl

_E = 64
_ROWS = 1024


def _fused_body(x_ref, wt_ref, rb_ref, eb_ref, out_ref):
    x = x_ref[...]
    logits = jnp.dot(x, wt_ref[...], preferred_element_type=jnp.float32)
    logits = logits + rb_ref[...]
    m = jnp.max(logits, axis=-1, keepdims=True)
    ex = jnp.exp(logits - m)
    scores = ex / jnp.sum(ex, axis=-1, keepdims=True)

    m1 = jnp.max(scores, axis=-1, keepdims=True)
    s2 = jnp.where(scores == m1, -jnp.inf, scores)
    m2 = jnp.max(s2, axis=-1, keepdims=True)
    s_masked = jnp.where(scores >= m2, scores, 0.0)

    comb = jnp.dot(s_masked.astype(jnp.bfloat16), eb_ref[...],
                   preferred_element_type=jnp.float32)
    out_ref[...] = (m1 + m2) * x + comb


def kernel(hidden_states, router_weight, router_bias, expert_bias):
    B, S, H = hidden_states.shape
    BS = B * S
    flat = hidden_states.reshape(BS, H)
    wt = router_weight.T
    rb = router_bias.reshape(1, _E)
    eb16 = expert_bias.astype(jnp.bfloat16)

    out = pl.pallas_call(
        _fused_body,
        grid=(BS // _ROWS,),
        in_specs=[
            pl.BlockSpec((_ROWS, H), lambda i: (i, 0)),
            pl.BlockSpec((H, _E), lambda i: (0, 0)),
            pl.BlockSpec((1, _E), lambda i: (0, 0)),
            pl.BlockSpec((_E, H), lambda i: (0, 0)),
        ],
        out_specs=pl.BlockSpec((_ROWS, H), lambda i: (i, 0)),
        out_shape=jax.ShapeDtypeStruct((BS, H), jnp.float32),
    )(flat, wt, rb, eb16)
    return out.reshape(B, S, H)


# speedup vs baseline: 11.3725x; 1.0114x over previous
import jax
import jax.numpy as jnp
from jax.experimental import pallas as pl

_E = 64
_ROWS = 1024


def _fused_body(x_ref, wt_ref, rb_ref, eb_ref, out_ref):
    x = x_ref[...]
    logits = jnp.dot(x, wt_ref[...], preferred_element_type=jnp.float32)
    logits = logits + rb_ref[...]

    ml1 = jnp.max(logits, axis=-1, keepdims=True)
    lm = jnp.where(logits == ml1, -jnp.inf, logits)
    ml2 = jnp.max(lm, axis=-1, keepdims=True)

    ex = jnp.exp(logits - ml1)
    r = 1.0 / jnp.sum(ex, axis=-1, keepdims=True)

    s_masked = jnp.where(logits >= ml2, ex, 0.0) * r
    wsum = (1.0 + jnp.exp(ml2 - ml1)) * r

    comb = jnp.dot(s_masked.astype(jnp.bfloat16), eb_ref[...],
                   preferred_element_type=jnp.float32)
    out_ref[...] = wsum * x + comb


def kernel(hidden_states, router_weight, router_bias, expert_bias):
    B, S, H = hidden_states.shape
    BS = B * S
    flat = hidden_states.reshape(BS, H)
    wt = router_weight.T
    rb = router_bias.reshape(1, _E)
    eb16 = expert_bias.astype(jnp.bfloat16)

    out = pl.pallas_call(
        _fused_body,
        grid=(BS // _ROWS,),
        in_specs=[
            pl.BlockSpec((_ROWS, H), lambda i: (i, 0)),
            pl.BlockSpec((H, _E), lambda i: (0, 0)),
            pl.BlockSpec((1, _E), lambda i: (0, 0)),
            pl.BlockSpec((_E, H), lambda i: (0, 0)),
        ],
        out_specs=pl.BlockSpec((_ROWS, H), lambda i: (i, 0)),
        out_shape=jax.ShapeDtypeStruct((BS, H), jnp.float32),
    )(flat, wt, rb, eb16)
    return out.reshape(B, S, H)
